# baseline (speedup 1.0000x reference)
import functools

import jax
import jax.numpy as jnp
from jax import lax
from jax.experimental import pallas as pl
from jax.experimental.pallas import tpu as pltpu
from jax.experimental.pallas import tpu_sc as plsc

NUM_DIVISION = 10
NUM_NODES = 100000
D_FEAT = 128
TAU = 1.0
CHUNK = NUM_NODES // NUM_DIVISION

TOTAL = NUM_NODES * D_FEAT
NUM_WORKERS = 32
BLK = 20_000
NBUF = 6
PREFETCH_D = 3
STRIDE = NUM_WORKERS * BLK
NUM_BLOCKS = TOTAL // (NUM_WORKERS * BLK)
VEC_ITERS = BLK // 16
IDX_PER_WORKER = 400
IDX_WORKERS = CHUNK // IDX_PER_WORKER

_MESH = plsc.VectorSubcoreMesh(core_axis_name="c", subcore_axis_name="s")


def _sc_body(idx_hbm, scale_hbm, x_hbm, out_hbm, outidx_hbm,
             idx_v, scale_v, oi_buf, b0, b1, b2, b3, b4, b5,
             in_sems, out_sems):
    bufs = (b0, b1, b2, b3, b4, b5)
    w = lax.axis_index("s") * 2 + lax.axis_index("c")

    pltpu.sync_copy(idx_hbm, idx_v)
    pltpu.sync_copy(scale_hbm, scale_v)
    idx = idx_v[...][0]
    sv = scale_v[...]

    def start_in(k):
        b = k % NBUF
        return pltpu.async_copy(
            x_hbm.at[pl.ds(w * BLK + k * STRIDE, BLK)], bufs[b], in_sems.at[b]
        )

    handles_in = [None] * NBUF
    handles_out = [None] * NBUF
    for k in range(PREFETCH_D):
        handles_in[k % NBUF] = start_in(k)

    @pl.when(w < IDX_WORKERS)
    def _():
        base = idx * CHUNK + w * IDX_PER_WORKER
        iota = lax.iota(jnp.int32, 16)

        def body(i, carry):
            oi_buf[pl.ds(i * 16, 16)] = base + i * 16 + iota
            return carry

        lax.fori_loop(0, IDX_PER_WORKER // 16, body, 0)
        pltpu.sync_copy(
            oi_buf, outidx_hbm.at[pl.ds(w * IDX_PER_WORKER, IDX_PER_WORKER)]
        )

    for k in range(NUM_BLOCKS):
        b = k % NBUF
        handles_in[b].wait()

        @pl.when(idx == (k // 2))
        def _():
            def body(i, carry):
                sl = pl.ds(i * 16, 16)
                bufs[b][sl] = bufs[b][sl] * sv
                return carry

            lax.fori_loop(0, VEC_ITERS, body, 0)

        handles_out[b] = pltpu.async_copy(
            bufs[b], out_hbm.at[pl.ds(w * BLK + k * STRIDE, BLK)], out_sems.at[b]
        )

        nk = k + PREFETCH_D
        if nk < NUM_BLOCKS:
            bb = nk % NBUF
            if handles_out[bb] is not None:
                handles_out[bb].wait()
            handles_in[bb] = start_in(nk)

    for k in range(max(0, NUM_BLOCKS - NBUF), NUM_BLOCKS):
        handles_out[k % NBUF].wait()


_sc_copy_scale = functools.partial(
    pl.kernel,
    out_type=[
        jax.ShapeDtypeStruct((TOTAL,), jnp.float32),
        jax.ShapeDtypeStruct((CHUNK,), jnp.int32),
    ],
    mesh=_MESH,
    scratch_types=[
        pltpu.VMEM((16,), jnp.int32),
        pltpu.VMEM((16,), jnp.float32),
        pltpu.VMEM((IDX_PER_WORKER,), jnp.int32),
        pltpu.VMEM((BLK,), jnp.float32),
        pltpu.VMEM((BLK,), jnp.float32),
        pltpu.VMEM((BLK,), jnp.float32),
        pltpu.VMEM((BLK,), jnp.float32),
        pltpu.VMEM((BLK,), jnp.float32),
        pltpu.VMEM((BLK,), jnp.float32),
        pltpu.SemaphoreType.DMA((NBUF,)),
        pltpu.SemaphoreType.DMA((NBUF,)),
    ],
)(_sc_body)


@jax.jit
def kernel(node_features, uniform_noise, sample_weights):
    g = -jnp.log(-jnp.log(uniform_noise))
    y_soft = jax.nn.softmax((sample_weights + g) / TAU, axis=-1)
    idx = jnp.argmax(y_soft, axis=-1).astype(jnp.int32)
    y = (1.0 + y_soft[idx]) - y_soft[idx]

    idx16 = jnp.full((16,), idx, dtype=jnp.int32)
    scale16 = jnp.full((16,), y, dtype=jnp.float32)
    x_flat = node_features.reshape(TOTAL)

    out_flat, outidx = _sc_copy_scale(idx16, scale16, x_flat)
    return out_flat.reshape(NUM_NODES, D_FEAT), outidx


# speedup vs baseline: 1.2538x; 1.2538x over previous
import functools

import jax
import jax.numpy as jnp
from jax import lax
from jax.experimental import pallas as pl
from jax.experimental.pallas import tpu as pltpu
from jax.experimental.pallas import tpu_sc as plsc

NUM_DIVISION = 10
NUM_NODES = 100000
D_FEAT = 128
TAU = 1.0
CHUNK = NUM_NODES // NUM_DIVISION

BLOCK_ROWS = 10000
NUM_BLOCKS = NUM_NODES // BLOCK_ROWS

IDX_PER_WORKER = 400
IDX_WORKERS = CHUNK // IDX_PER_WORKER

_MESH = plsc.VectorSubcoreMesh(core_axis_name="c", subcore_axis_name="s")


def _copy_scale_kernel(idx_ref, scale_ref, x_ref, out_ref):
    i = pl.program_id(0)
    row0 = i * BLOCK_ROWS
    rows = row0 + jax.lax.broadcasted_iota(jnp.int32, (BLOCK_ROWS, 1), 0)
    lo = idx_ref[0] * CHUNK
    in_slab = (rows >= lo) & (rows < lo + CHUNK)
    w = jnp.where(in_slab, scale_ref[0], jnp.float32(1.0))
    out_ref[...] = x_ref[...] * w


def _sc_indices_body(idx_hbm, outidx_hbm, idx_v, oi_buf):
    w = lax.axis_index("s") * 2 + lax.axis_index("c")

    @pl.when(w < IDX_WORKERS)
    def _():
        pltpu.sync_copy(idx_hbm, idx_v)
        idx = idx_v[...][0]
        base = idx * CHUNK + w * IDX_PER_WORKER
        iota = lax.iota(jnp.int32, 16)

        def body(i, carry):
            oi_buf[pl.ds(i * 16, 16)] = base + i * 16 + iota
            return carry

        lax.fori_loop(0, IDX_PER_WORKER // 16, body, 0)
        pltpu.sync_copy(
            oi_buf, outidx_hbm.at[pl.ds(w * IDX_PER_WORKER, IDX_PER_WORKER)]
        )


_sc_indices = functools.partial(
    pl.kernel,
    out_type=jax.ShapeDtypeStruct((CHUNK,), jnp.int32),
    mesh=_MESH,
    scratch_types=[
        pltpu.VMEM((16,), jnp.int32),
        pltpu.VMEM((IDX_PER_WORKER,), jnp.int32),
    ],
)(_sc_indices_body)


@jax.jit
def kernel(node_features, uniform_noise, sample_weights):
    g = -jnp.log(-jnp.log(uniform_noise))
    y_soft = jax.nn.softmax((sample_weights + g) / TAU, axis=-1)
    idx = jnp.argmax(y_soft, axis=-1).astype(jnp.int32)
    y = (1.0 + y_soft[idx]) - y_soft[idx]
    idx_arr = idx[None]
    scale_arr = y[None].astype(jnp.float32)
    idx16 = jnp.full((16,), idx, dtype=jnp.int32)

    updated = pl.pallas_call(
        _copy_scale_kernel,
        grid=(NUM_BLOCKS,),
        in_specs=[
            pl.BlockSpec(memory_space=pltpu.SMEM),
            pl.BlockSpec(memory_space=pltpu.SMEM),
            pl.BlockSpec((BLOCK_ROWS, D_FEAT), lambda i: (i, 0)),
        ],
        out_specs=pl.BlockSpec((BLOCK_ROWS, D_FEAT), lambda i: (i, 0)),
        out_shape=jax.ShapeDtypeStruct((NUM_NODES, D_FEAT), jnp.float32),
        compiler_params=pltpu.CompilerParams(
            dimension_semantics=("arbitrary",),
        ),
    )(idx_arr, scale_arr, node_features)

    outidx = _sc_indices(idx16)
    return updated, outidx


# speedup vs baseline: 1.2755x; 1.0173x over previous
import functools

import jax
import jax.numpy as jnp
from jax import lax
from jax.experimental import pallas as pl
from jax.experimental.pallas import tpu as pltpu
from jax.experimental.pallas import tpu_sc as plsc

NUM_DIVISION = 10
NUM_NODES = 100000
D_FEAT = 128
TAU = 1.0
CHUNK = NUM_NODES // NUM_DIVISION

BLOCK_ROWS = 10000
NUM_BLOCKS = NUM_NODES // BLOCK_ROWS

IDX_PER_WORKER = 400
IDX_WORKERS = CHUNK // IDX_PER_WORKER

_MESH = plsc.VectorSubcoreMesh(core_axis_name="c", subcore_axis_name="s")


BLOCKS_PER_CHUNK = max(1, CHUNK // BLOCK_ROWS)


def _copy_scale_kernel(idx_ref, scale_ref, x_ref, out_ref):
    i = pl.program_id(0)
    in_slab = (i // BLOCKS_PER_CHUNK) == idx_ref[0]
    w = jnp.where(in_slab, scale_ref[0], jnp.float32(1.0))
    out_ref[...] = x_ref[...] * w


def _sc_indices_body(idx_hbm, outidx_hbm, idx_v, oi_buf):
    w = lax.axis_index("s") * 2 + lax.axis_index("c")

    @pl.when(w < IDX_WORKERS)
    def _():
        pltpu.sync_copy(idx_hbm, idx_v)
        idx = idx_v[...][0]
        base = idx * CHUNK + w * IDX_PER_WORKER
        iota = lax.iota(jnp.int32, 16)

        def body(i, carry):
            oi_buf[pl.ds(i * 16, 16)] = base + i * 16 + iota
            return carry

        lax.fori_loop(0, IDX_PER_WORKER // 16, body, 0)
        pltpu.sync_copy(
            oi_buf, outidx_hbm.at[pl.ds(w * IDX_PER_WORKER, IDX_PER_WORKER)]
        )


_sc_indices = functools.partial(
    pl.kernel,
    out_type=jax.ShapeDtypeStruct((CHUNK,), jnp.int32),
    mesh=_MESH,
    scratch_types=[
        pltpu.VMEM((16,), jnp.int32),
        pltpu.VMEM((IDX_PER_WORKER,), jnp.int32),
    ],
)(_sc_indices_body)


@jax.jit
def kernel(node_features, uniform_noise, sample_weights):
    g = -jnp.log(-jnp.log(uniform_noise))
    y_soft = jax.nn.softmax((sample_weights + g) / TAU, axis=-1)
    idx = jnp.argmax(y_soft, axis=-1).astype(jnp.int32)
    y = (1.0 + y_soft[idx]) - y_soft[idx]
    idx_arr = idx[None]
    scale_arr = y[None].astype(jnp.float32)
    idx16 = jnp.full((16,), idx, dtype=jnp.int32)

    updated = pl.pallas_call(
        _copy_scale_kernel,
        grid=(NUM_BLOCKS,),
        in_specs=[
            pl.BlockSpec(memory_space=pltpu.SMEM),
            pl.BlockSpec(memory_space=pltpu.SMEM),
            pl.BlockSpec((BLOCK_ROWS, D_FEAT), lambda i: (i, 0)),
        ],
        out_specs=pl.BlockSpec((BLOCK_ROWS, D_FEAT), lambda i: (i, 0)),
        out_shape=jax.ShapeDtypeStruct((NUM_NODES, D_FEAT), jnp.float32),
        compiler_params=pltpu.CompilerParams(
            dimension_semantics=("arbitrary",),
        ),
    )(idx_arr, scale_arr, node_features)

    outidx = _sc_indices(idx16)
    return updated, outidx
